# baseline (speedup 1.0000x reference)
import functools
import math

import jax
import jax.numpy as jnp
from jax import lax
from jax.experimental import pallas as pl
from jax.experimental.pallas import tpu as pltpu
from jax.experimental.pallas import tpu_sc as plsc

N2 = 10000
N3 = 50000
N2P = 10240
N3P = 50176
E2 = 320000
E3 = 800000
CH2 = 80
CH3 = 196
E2P = CH2 * 128 * 32
E3P = CH3 * 128 * 32
CHG = 14
BQ = CHG * 128 * 32
EPS = 1e-5
ZR = 32


@functools.lru_cache(maxsize=None)
def _make_sc_agg(n_chunks, gw, n_pad, w, nb, nr):
    rows_sc_tile = nr * n_pad // 16
    nw = n_chunks // gw
    mesh = plsc.VectorSubcoreMesh(core_axis_name="c", subcore_axis_name="s")

    @functools.partial(
        pl.kernel,
        out_type=jax.ShapeDtypeStruct((2, nr * n_pad, w), jnp.float32),
        mesh=mesh,
        scratch_types=[
            pltpu.VMEM((gw, 128), jnp.int32),
            pltpu.VMEM((gw, 128), jnp.int32),
            pltpu.VMEM((nb, 128, w), jnp.float32),
            pltpu.VMEM((ZR, w), jnp.float32),
            pltpu.VMEM_SHARED((nr * n_pad, w), jnp.float32),
        ] + [pltpu.SemaphoreType.DMA] * nb,
        compiler_params=pltpu.CompilerParams(use_tc_tiling_on_sc=False),
    )
    def agg(h_hbm, src_hbm, dst_hbm, out_hbm, src_w, dst_w, rows_v, zero_v,
            acc, *sems):
        c = lax.axis_index("c")
        s = lax.axis_index("s")
        wid = s * 2 + c
        rep = lax.rem(s, nr)
        for i in range(ZR):
            for j in range(w // 16):
                zero_v[i, pl.ds(j * 16, 16)] = jnp.zeros((16,), jnp.float32)
        zbase = s * rows_sc_tile

        def zloop(r, carry):
            pltpu.sync_copy(zero_v, acc.at[pl.ds(zbase + r * ZR, ZR)])
            return carry

        lax.fori_loop(0, rows_sc_tile // ZR, zloop, 0)
        plsc.subcore_barrier()

        def window(w_i, carry):
            base = wid * n_chunks + w_i * gw
            pltpu.sync_copy(src_hbm.at[pl.ds(base, gw)], src_w)
            pltpu.sync_copy(dst_hbm.at[rep, pl.ds(base, gw)], dst_w)
            for b in range(nb):
                pltpu.make_async_copy(h_hbm.at[src_w.at[b]], rows_v.at[b],
                                      sems[b]).start()
            for jj in range(gw):
                b = jj % nb
                pltpu.make_async_copy(h_hbm.at[src_w.at[jj]], rows_v.at[b],
                                      sems[b]).wait()
                pltpu.sync_copy(rows_v.at[b], acc.at[dst_w.at[jj]], add=True)
                if jj + nb < gw:
                    pltpu.make_async_copy(h_hbm.at[src_w.at[jj + nb]],
                                          rows_v.at[b], sems[b]).start()
            return carry

        lax.fori_loop(0, nw, window, 0)
        plsc.subcore_barrier()
        pltpu.sync_copy(acc.at[pl.ds(zbase, rows_sc_tile)],
                        out_hbm.at[c, pl.ds(zbase, rows_sc_tile)])

    return agg


@functools.lru_cache(maxsize=None)
def _make_sc_gather(n_chunks, w, nb):
    mesh = plsc.VectorSubcoreMesh(core_axis_name="c", subcore_axis_name="s")

    @functools.partial(
        pl.kernel,
        out_type=jax.ShapeDtypeStruct((32 * n_chunks * 128, w), jnp.float32),
        mesh=mesh,
        scratch_types=[
            pltpu.VMEM((n_chunks, 128), jnp.int32),
            pltpu.VMEM((nb, 128, w), jnp.float32),
        ] + [pltpu.SemaphoreType.DMA] * nb,
        compiler_params=pltpu.CompilerParams(use_tc_tiling_on_sc=False),
    )
    def gather(table_hbm, idx_hbm, out_hbm, idx_v, rows_v, *sems):
        c = lax.axis_index("c")
        s = lax.axis_index("s")
        wid = s * 2 + c
        pltpu.sync_copy(idx_hbm.at[pl.ds(wid * n_chunks, n_chunks)], idx_v)
        for b in range(nb):
            pltpu.make_async_copy(table_hbm.at[idx_v.at[b]], rows_v.at[b],
                                  sems[b]).start()
        rbase = wid * n_chunks * 128

        def group(jg, carry):
            for b in range(nb):
                j = jg * nb + b
                pltpu.make_async_copy(table_hbm.at[idx_v.at[j]], rows_v.at[b],
                                      sems[b]).wait()
                pltpu.sync_copy(rows_v.at[b],
                                out_hbm.at[pl.ds(rbase + j * 128, 128)])

                @pl.when(jg + 1 < n_chunks // nb)
                def _issue():
                    pltpu.make_async_copy(table_hbm.at[idx_v.at[j + nb]],
                                          rows_v.at[b], sems[b]).start()
            return carry

        lax.fori_loop(0, n_chunks // nb, group, 0)

    return gather


def _elu(z):
    return jnp.where(z > 0, z, jnp.exp(jnp.minimum(z, 0.0)) - 1.0)


def _bn_full(z, gamma, beta):
    mu = jnp.mean(z, axis=0, keepdims=True)
    var = jnp.mean(z * z, axis=0, keepdims=True) - mu * mu
    return (z - mu) * lax.rsqrt(var + EPS) * gamma + beta


def _tc(fn, out_shapes, *arrs):
    n_in = len(arrs)

    def body(*refs):
        outs = fn(*[r[...] for r in refs[:n_in]])
        if not isinstance(outs, (tuple, list)):
            outs = (outs,)
        for r, v in zip(refs[n_in:], outs):
            r[...] = v

    res = pl.pallas_call(
        body,
        out_shape=[jax.ShapeDtypeStruct(s, jnp.float32) for s in out_shapes],
        compiler_params=pltpu.CompilerParams(vmem_limit_bytes=100 * 2**20),
    )(*arrs)
    return res if len(out_shapes) > 1 else res[0]


def _sum2(sp):
    total = None
    for ci in range(2):
        for r in range(sp.shape[1] // N2P):
            part = sp[ci, r * N2P:r * N2P + N2]
            total = part if total is None else total + part
    return total


R3 = 1000
G3 = N3 // R3


def _g3_z_stats(slo, shi, xin, wroot, b):
    def body(slo_ref, shi_ref, xin_ref, wr_ref, b_ref, z_ref, st_ref, acc):
        i = pl.program_id(0)
        s = jnp.concatenate([slo_ref[0] + slo_ref[1],
                             shi_ref[0] + shi_ref[1]], axis=1)
        z = s + jnp.dot(xin_ref[...], wr_ref[...],
                        preferred_element_type=jnp.float32) + b_ref[...]
        z_ref[...] = z

        @pl.when(i == 0)
        def _init():
            acc[...] = jnp.zeros_like(acc)

        acc[0:1] += jnp.sum(z, axis=0, keepdims=True)
        acc[1:2] += jnp.sum(z * z, axis=0, keepdims=True)
        st_ref[...] = acc[...]

    return pl.pallas_call(
        body,
        grid=(G3,),
        in_specs=[
            pl.BlockSpec((2, R3, 32), lambda i: (0, i, 0)),
            pl.BlockSpec((2, R3, 32), lambda i: (0, i, 0)),
            pl.BlockSpec((R3, 64), lambda i: (i, 0)),
            pl.BlockSpec((64, 64), lambda i: (0, 0)),
            pl.BlockSpec((1, 64), lambda i: (0, 0)),
        ],
        out_specs=[
            pl.BlockSpec((R3, 64), lambda i: (i, 0)),
            pl.BlockSpec((2, 64), lambda i: (0, 0)),
        ],
        out_shape=[
            jax.ShapeDtypeStruct((N3, 64), jnp.float32),
            jax.ShapeDtypeStruct((2, 64), jnp.float32),
        ],
        scratch_shapes=[pltpu.VMEM((2, 64), jnp.float32)],
    )(slo, shi, xin, wroot, b)


def _g3_bn_act(z, stats, gamma, beta, wlo, whi):
    def body(z_ref, st_ref, g_ref, bt_ref, wlo_ref, whi_ref,
             a_ref, plo_ref, phi_ref):
        st = st_ref[...]
        mu = st[0:1] * (1.0 / N3)
        var = st[1:2] * (1.0 / N3) - mu * mu
        a = _elu((z_ref[...] - mu) * lax.rsqrt(var + EPS) * g_ref[...]
                 + bt_ref[...])
        a_ref[...] = a
        plo_ref[...] = jnp.dot(a, wlo_ref[...],
                               preferred_element_type=jnp.float32)
        phi_ref[...] = jnp.dot(a, whi_ref[...],
                               preferred_element_type=jnp.float32)

    return pl.pallas_call(
        body,
        grid=(G3,),
        in_specs=[
            pl.BlockSpec((R3, 64), lambda i: (i, 0)),
            pl.BlockSpec((2, 64), lambda i: (0, 0)),
            pl.BlockSpec((1, 64), lambda i: (0, 0)),
            pl.BlockSpec((1, 64), lambda i: (0, 0)),
            pl.BlockSpec((64, 32), lambda i: (0, 0)),
            pl.BlockSpec((64, 32), lambda i: (0, 0)),
        ],
        out_specs=[
            pl.BlockSpec((R3, 64), lambda i: (i, 0)),
            pl.BlockSpec((R3, 32), lambda i: (i, 0)),
            pl.BlockSpec((R3, 32), lambda i: (i, 0)),
        ],
        out_shape=[
            jax.ShapeDtypeStruct((N3, 64), jnp.float32),
            jax.ShapeDtypeStruct((N3, 32), jnp.float32),
            jax.ShapeDtypeStruct((N3, 32), jnp.float32),
        ],
    )(z, stats, gamma, beta, wlo, whi)


def _g3_proj(xin, wlo, whi):
    def body(x_ref, wlo_ref, whi_ref, plo_ref, phi_ref):
        xv = x_ref[...]
        plo_ref[...] = jnp.dot(xv, wlo_ref[...],
                               preferred_element_type=jnp.float32)
        phi_ref[...] = jnp.dot(xv, whi_ref[...],
                               preferred_element_type=jnp.float32)

    return pl.pallas_call(
        body,
        grid=(G3,),
        in_specs=[
            pl.BlockSpec((R3, 64), lambda i: (i, 0)),
            pl.BlockSpec((64, 32), lambda i: (0, 0)),
            pl.BlockSpec((64, 32), lambda i: (0, 0)),
        ],
        out_specs=[
            pl.BlockSpec((R3, 32), lambda i: (i, 0)),
            pl.BlockSpec((R3, 32), lambda i: (i, 0)),
        ],
        out_shape=[
            jax.ShapeDtypeStruct((N3, 32), jnp.float32),
            jax.ShapeDtypeStruct((N3, 32), jnp.float32),
        ],
    )(xin, wlo, whi)


def _g3_final(slo, shi, xin, wrel, wroot, b):
    def body(slo_ref, shi_ref, xin_ref, wrel_ref, wroot_ref, b_ref, o_ref):
        s = jnp.concatenate([slo_ref[0] + slo_ref[1],
                             shi_ref[0] + shi_ref[1]], axis=1)
        o_ref[...] = (jnp.dot(s, wrel_ref[...],
                              preferred_element_type=jnp.float32)
                      + jnp.dot(xin_ref[...], wroot_ref[...],
                                preferred_element_type=jnp.float32)
                      + b_ref[...])

    return pl.pallas_call(
        body,
        grid=(G3,),
        in_specs=[
            pl.BlockSpec((2, R3, 32), lambda i: (0, i, 0)),
            pl.BlockSpec((2, R3, 32), lambda i: (0, i, 0)),
            pl.BlockSpec((R3, 64), lambda i: (i, 0)),
            pl.BlockSpec((64, 128), lambda i: (0, 0)),
            pl.BlockSpec((64, 128), lambda i: (0, 0)),
            pl.BlockSpec((1, 128), lambda i: (0, 0)),
        ],
        out_specs=pl.BlockSpec((R3, 128), lambda i: (i, 0)),
        out_shape=jax.ShapeDtypeStruct((N3, 128), jnp.float32),
    )(slo, shi, xin, wrel, wroot, b)


QB = 512


def _knn(px_t, py):
    def body(px_ref, py_ref, o_ref):
        pyv = py_ref[...]
        d2 = jnp.zeros((QB, N2), jnp.float32)
        for k in range(3):
            diff = pyv[:, k:k + 1] - px_ref[k:k + 1, :]
            d2 = d2 + diff * diff
        m = jnp.min(d2, axis=1, keepdims=True)
        io = lax.broadcasted_iota(jnp.int32, (QB, N2), 1)
        sel = jnp.where(d2 == m, io, jnp.int32(N2))
        o_ref[...] = jnp.min(sel, axis=1, keepdims=True)

    return pl.pallas_call(
        body,
        grid=(pl.cdiv(N3, QB),),
        in_specs=[
            pl.BlockSpec((3, N2), lambda i: (0, 0)),
            pl.BlockSpec((QB, 3), lambda i: (i, 0)),
        ],
        out_specs=pl.BlockSpec((QB, 1), lambda i: (i, 0)),
        out_shape=jax.ShapeDtypeStruct((N3, 1), jnp.int32),
        compiler_params=pltpu.CompilerParams(vmem_limit_bytes=100 * 2**20),
    )(px_t, py)


def _onera(pos):
    p0 = pos[:, 0] - math.tan(math.pi / 6) * pos[:, 1]
    pos = jnp.concatenate([p0[:, None], pos[:, 1:]], axis=1)
    return pos * (1 + (1 / 0.56 - 1) * (pos[:, 1:2] / 1.1963))


def _edge_prep(ei, e, ep, npad, nr, n):
    src = jnp.pad(ei[0].astype(jnp.int32), (0, ep - e)).reshape(-1, 128)
    pad_dst = n + jnp.arange(ep - e, dtype=jnp.int32) % (npad - n)
    dst1 = jnp.concatenate([ei[1].astype(jnp.int32), pad_dst])
    dst = jnp.stack([(dst1 + r * npad).reshape(-1, 128) for r in range(nr)])
    return src, dst


def kernel(x, edge_index_2, edge_index_3, pos_2, pos_3, y, params):
    src2, dst2 = _edge_prep(edge_index_2, E2, E2P, N2P, 1, N2)
    src3, dst3 = _edge_prep(edge_index_3, E3, E3P, N3P, 1, N3)
    agg2_32 = _make_sc_agg(CH2, 8, N2P, 32, 8, 1)
    agg2_64 = _make_sc_agg(CH2, 8, N2P, 64, 8, 1)
    agg3_32 = _make_sc_agg(CH3, 7, N3P, 32, 4, 1)

    p = params
    row = lambda a: a.reshape(1, -1)

    wr, wo, b1 = p['conv1']['Wrel'], p['conv1']['Wroot'], p['conv1']['b']
    y2 = y.reshape(1, 1)

    def k0(xv, p2v, yv, wra, wrp, wry, woa, wop, woy):
        def mix(wa, wp, wy):
            t = jnp.dot(xv, wa, preferred_element_type=jnp.float32)
            for k in range(3):
                t = t + p2v[:, k:k + 1] * wp[k:k + 1, :]
            return t + yv * wy
        return mix(wra, wrp, wry), mix(woa, wop, woy)

    proj1, root1 = _tc(
        k0, [(N2, 64), (N2, 64)], x, pos_2, y2,
        wr[:128], wr[128:131], row(wr[131:].sum(0)),
        wo[:128], wo[128:131], row(wo[131:].sum(0)))

    sp = agg2_64(proj1, src2, dst2)

    wn_first = p['erdb1']['rdb1']['conv1']['Wrel']

    def k1(spv, rootv, bv, gv, btv, wnv):
        x1 = _bn_full(_sum2(spv) + bv + rootv, gv, btv)
        return x1, jnp.dot(x1, wnv, preferred_element_type=jnp.float32)

    x1, pj = _tc(k1, [(N2, 64), (N2, 32)], sp, root1, row(b1),
                 row(p['bn1']['gamma']), row(p['bn1']['beta']), wn_first)

    def rdb_run(pr, r0, pj_in, tail, wnext):
        sp1 = agg2_32(pj_in, src2, dst2)
        c1, c2, c3 = pr['conv1'], pr['conv2'], pr['conv3']

        def ka(spv, r0v, bv, wov, gv, btv, w2a, w2b):
            t1 = _elu(_bn_full(
                _sum2(spv) + bv
                + jnp.dot(r0v, wov, preferred_element_type=jnp.float32),
                gv, btv))
            pn = (jnp.dot(r0v, w2a, preferred_element_type=jnp.float32)
                  + jnp.dot(t1, w2b, preferred_element_type=jnp.float32))
            return t1, pn

        t1, pj2 = _tc(ka, [(N2, 32), (N2, 32)], sp1, r0, row(c1['b']),
                      c1['Wroot'], row(pr['bn1']['gamma']),
                      row(pr['bn1']['beta']),
                      c2['Wrel'][:64], c2['Wrel'][64:])

        sp2 = agg2_32(pj2, src2, dst2)

        def kb(spv, r0v, t1v, bv, woa, wob, gv, btv, w3a, w3b, w3c):
            t2 = _elu(_bn_full(
                _sum2(spv) + bv
                + jnp.dot(r0v, woa, preferred_element_type=jnp.float32)
                + jnp.dot(t1v, wob, preferred_element_type=jnp.float32),
                gv, btv))
            pn = (jnp.dot(r0v, w3a, preferred_element_type=jnp.float32)
                  + jnp.dot(t1v, w3b, preferred_element_type=jnp.float32)
                  + jnp.dot(t2, w3c, preferred_element_type=jnp.float32))
            return t2, pn

        w3 = c3['Wrel']
        t2, pj3 = _tc(
            kb, [(N2, 32), (N2, 64)], sp2, r0, t1, row(c2['b']),
            c2['Wroot'][:64], c2['Wroot'][64:],
            row(pr['bn2']['gamma']), row(pr['bn2']['beta']),
            w3[:64], w3[64:96], w3[96:])

        sp3 = agg2_64(pj3, src2, dst2)
        kind, extras = tail

        def kc(spv, r0v, t1v, t2v, bv, woa, wob, woc, gv, btv, *rest):
            t3 = _elu(_bn_full(
                _sum2(spv) + bv
                + jnp.dot(r0v, woa, preferred_element_type=jnp.float32)
                + jnp.dot(t1v, wob, preferred_element_type=jnp.float32)
                + jnp.dot(t2v, woc, preferred_element_type=jnp.float32),
                gv, btv))
            rdbout = r0v + 0.2 * t3
            if kind == 'rdb1':
                (wnv,) = rest
                carry = r0v + 0.2 * rdbout
            elif kind == 'rdb2':
                a0v, wnv = rest
                carry = a0v + 0.2 * rdbout
            else:
                a0v, x1v = rest
                carry = x1v + 0.2 * (a0v + 0.2 * rdbout)
            if kind == 'final':
                return (carry,)
            return carry, jnp.dot(carry, wnv,
                                  preferred_element_type=jnp.float32)

        common = [sp3, r0, t1, t2, row(c3['b']),
                  c3['Wroot'][:64], c3['Wroot'][64:96], c3['Wroot'][96:],
                  row(pr['bn3']['gamma']), row(pr['bn3']['beta'])]
        if kind == 'final':
            (out,) = _tc(kc, [(N2, 64)], *common, *extras),
            return out, None
        res = _tc(kc, [(N2, 64), (N2, 32)], *common, *extras, wnext)
        return res[0], res[1]

    e1, e2 = p['erdb1'], p['erdb2']
    r0b, pj = rdb_run(e1['rdb1'], x1, pj, ('rdb1', []),
                      e1['rdb2']['conv1']['Wrel'])
    x2, pj = rdb_run(e1['rdb2'], r0b, pj, ('rdb2', [x1]),
                     e2['rdb1']['conv1']['Wrel'])
    r0b, pj = rdb_run(e2['rdb1'], x2, pj, ('rdb1', []),
                      e2['rdb2']['conv1']['Wrel'])
    x3, _ = rdb_run(e2['rdb2'], r0b, pj, ('final', [x2, x1]), None)

    px = _onera(pos_2)
    py = _onera(pos_3)
    idx = _knn(px.T, py)[:, 0]
    idxm = jnp.pad(idx, (0, BQ - N3)).reshape(-1, 128)
    xi = _make_sc_gather(CHG, 64, 2)(x3, idxm)[:N3]

    c3p, c4p, c5p = p['conv3'], p['conv4'], p['conv5']
    plo, phi = _g3_proj(xi, c3p['Wrel'][:, :32], c3p['Wrel'][:, 32:])
    slo = agg3_32(plo, src3, dst3)
    shi = agg3_32(phi, src3, dst3)
    z, st = _g3_z_stats(slo, shi, xi, c3p['Wroot'], row(c3p['b']))
    a3, plo, phi = _g3_bn_act(z, st, row(p['bn3']['gamma']),
                              row(p['bn3']['beta']),
                              c4p['Wrel'][:, :32], c4p['Wrel'][:, 32:])
    slo = agg3_32(plo, src3, dst3)
    shi = agg3_32(phi, src3, dst3)
    z, st = _g3_z_stats(slo, shi, a3, c4p['Wroot'], row(c4p['b']))
    eye = jnp.eye(64, dtype=jnp.float32)
    a4, alo, ahi = _g3_bn_act(z, st, row(p['bn4']['gamma']),
                              row(p['bn4']['beta']),
                              eye[:, :32], eye[:, 32:])
    slo = agg3_32(alo, src3, dst3)
    shi = agg3_32(ahi, src3, dst3)
    return _g3_final(slo, shi, a4, c5p['Wrel'], c5p['Wroot'], row(c5p['b']))


# speedup vs baseline: 1.0226x; 1.0226x over previous
import functools
import math

import jax
import jax.numpy as jnp
from jax import lax
from jax.experimental import pallas as pl
from jax.experimental.pallas import tpu as pltpu
from jax.experimental.pallas import tpu_sc as plsc

N2 = 10000
N3 = 50000
N2P = 10240
N3P = 50176
E2 = 320000
E3 = 800000
CH2 = 80
CH3 = 196
E2P = CH2 * 128 * 32
E3P = CH3 * 128 * 32
CHG = 14
BQ = CHG * 128 * 32
EPS = 1e-5
ZR = 32


@functools.lru_cache(maxsize=None)
def _make_sc_agg(n_chunks, gw, n_pad, w, nb, nr):
    rows_sc_tile = nr * n_pad // 16
    nw = n_chunks // gw
    mesh = plsc.VectorSubcoreMesh(core_axis_name="c", subcore_axis_name="s")

    @functools.partial(
        pl.kernel,
        out_type=jax.ShapeDtypeStruct((2, nr * n_pad, w), jnp.float32),
        mesh=mesh,
        scratch_types=[
            pltpu.VMEM((gw, 128), jnp.int32),
            pltpu.VMEM((gw, 128), jnp.int32),
            pltpu.VMEM((nb, 128, w), jnp.float32),
            pltpu.VMEM((ZR, w), jnp.float32),
            pltpu.VMEM_SHARED((nr * n_pad, w), jnp.float32),
        ] + [pltpu.SemaphoreType.DMA] * nb,
        compiler_params=pltpu.CompilerParams(use_tc_tiling_on_sc=False),
    )
    def agg(h_hbm, src_hbm, dst_hbm, out_hbm, src_w, dst_w, rows_v, zero_v,
            acc, *sems):
        c = lax.axis_index("c")
        s = lax.axis_index("s")
        wid = s * 2 + c
        rep = lax.rem(s, nr)
        for i in range(ZR):
            for j in range(w // 16):
                zero_v[i, pl.ds(j * 16, 16)] = jnp.zeros((16,), jnp.float32)
        zbase = s * rows_sc_tile

        def zloop(r, carry):
            pltpu.sync_copy(zero_v, acc.at[pl.ds(zbase + r * ZR, ZR)])
            return carry

        lax.fori_loop(0, rows_sc_tile // ZR, zloop, 0)
        plsc.subcore_barrier()

        def window(w_i, carry):
            base = wid * n_chunks + w_i * gw
            pltpu.sync_copy(src_hbm.at[pl.ds(base, gw)], src_w)
            pltpu.sync_copy(dst_hbm.at[rep, pl.ds(base, gw)], dst_w)
            for b in range(nb):
                pltpu.make_async_copy(h_hbm.at[src_w.at[b]], rows_v.at[b],
                                      sems[b]).start()
            for jj in range(gw):
                b = jj % nb
                pltpu.make_async_copy(h_hbm.at[src_w.at[jj]], rows_v.at[b],
                                      sems[b]).wait()
                pltpu.sync_copy(rows_v.at[b], acc.at[dst_w.at[jj]], add=True)
                if jj + nb < gw:
                    pltpu.make_async_copy(h_hbm.at[src_w.at[jj + nb]],
                                          rows_v.at[b], sems[b]).start()
            return carry

        lax.fori_loop(0, nw, window, 0)
        plsc.subcore_barrier()
        pltpu.sync_copy(acc.at[pl.ds(zbase, rows_sc_tile)],
                        out_hbm.at[c, pl.ds(zbase, rows_sc_tile)])

    return agg


@functools.lru_cache(maxsize=None)
def _make_sc_gather(n_chunks, w, nb):
    mesh = plsc.VectorSubcoreMesh(core_axis_name="c", subcore_axis_name="s")

    @functools.partial(
        pl.kernel,
        out_type=jax.ShapeDtypeStruct((32 * n_chunks * 128, w), jnp.float32),
        mesh=mesh,
        scratch_types=[
            pltpu.VMEM((n_chunks, 128), jnp.int32),
            pltpu.VMEM((nb, 128, w), jnp.float32),
        ] + [pltpu.SemaphoreType.DMA] * nb,
        compiler_params=pltpu.CompilerParams(use_tc_tiling_on_sc=False),
    )
    def gather(table_hbm, idx_hbm, out_hbm, idx_v, rows_v, *sems):
        c = lax.axis_index("c")
        s = lax.axis_index("s")
        wid = s * 2 + c
        pltpu.sync_copy(idx_hbm.at[pl.ds(wid * n_chunks, n_chunks)], idx_v)
        for b in range(nb):
            pltpu.make_async_copy(table_hbm.at[idx_v.at[b]], rows_v.at[b],
                                  sems[b]).start()
        rbase = wid * n_chunks * 128

        def group(jg, carry):
            for b in range(nb):
                j = jg * nb + b
                pltpu.make_async_copy(table_hbm.at[idx_v.at[j]], rows_v.at[b],
                                      sems[b]).wait()
                pltpu.sync_copy(rows_v.at[b],
                                out_hbm.at[pl.ds(rbase + j * 128, 128)])

                @pl.when(jg + 1 < n_chunks // nb)
                def _issue():
                    pltpu.make_async_copy(table_hbm.at[idx_v.at[j + nb]],
                                          rows_v.at[b], sems[b]).start()
            return carry

        lax.fori_loop(0, n_chunks // nb, group, 0)

    return gather


def _elu(z):
    return jnp.where(z > 0, z, jnp.exp(jnp.minimum(z, 0.0)) - 1.0)


def _bn_full(z, gamma, beta):
    mu = jnp.mean(z, axis=0, keepdims=True)
    var = jnp.mean(z * z, axis=0, keepdims=True) - mu * mu
    return (z - mu) * lax.rsqrt(var + EPS) * gamma + beta


def _tc(fn, out_shapes, *arrs):
    n_in = len(arrs)

    def body(*refs):
        outs = fn(*[r[...] for r in refs[:n_in]])
        if not isinstance(outs, (tuple, list)):
            outs = (outs,)
        for r, v in zip(refs[n_in:], outs):
            r[...] = v

    res = pl.pallas_call(
        body,
        out_shape=[jax.ShapeDtypeStruct(s, jnp.float32) for s in out_shapes],
        compiler_params=pltpu.CompilerParams(vmem_limit_bytes=100 * 2**20),
    )(*arrs)
    return res if len(out_shapes) > 1 else res[0]


def _sum2(sp):
    total = None
    for ci in range(2):
        for r in range(sp.shape[1] // N2P):
            part = sp[ci, r * N2P:r * N2P + N2]
            total = part if total is None else total + part
    return total


R3 = 1000
G3 = N3 // R3


def _g3_z_stats(slo, shi, xin, wroot, b):
    def body(slo_ref, shi_ref, xin_ref, wr_ref, b_ref, z_ref, st_ref, acc):
        i = pl.program_id(0)
        s = jnp.concatenate([slo_ref[0] + slo_ref[1],
                             shi_ref[0] + shi_ref[1]], axis=1)
        z = s + jnp.dot(xin_ref[...], wr_ref[...],
                        preferred_element_type=jnp.float32) + b_ref[...]
        z_ref[...] = z

        @pl.when(i == 0)
        def _init():
            acc[...] = jnp.zeros_like(acc)

        acc[0:1] += jnp.sum(z, axis=0, keepdims=True)
        acc[1:2] += jnp.sum(z * z, axis=0, keepdims=True)
        st_ref[...] = acc[...]

    return pl.pallas_call(
        body,
        grid=(G3,),
        in_specs=[
            pl.BlockSpec((2, R3, 32), lambda i: (0, i, 0)),
            pl.BlockSpec((2, R3, 32), lambda i: (0, i, 0)),
            pl.BlockSpec((R3, 64), lambda i: (i, 0)),
            pl.BlockSpec((64, 64), lambda i: (0, 0)),
            pl.BlockSpec((1, 64), lambda i: (0, 0)),
        ],
        out_specs=[
            pl.BlockSpec((R3, 64), lambda i: (i, 0)),
            pl.BlockSpec((2, 64), lambda i: (0, 0)),
        ],
        out_shape=[
            jax.ShapeDtypeStruct((N3, 64), jnp.float32),
            jax.ShapeDtypeStruct((2, 64), jnp.float32),
        ],
        scratch_shapes=[pltpu.VMEM((2, 64), jnp.float32)],
    )(slo, shi, xin, wroot, b)


def _g3_bn_act(z, stats, gamma, beta, wlo, whi):
    def body(z_ref, st_ref, g_ref, bt_ref, wlo_ref, whi_ref,
             a_ref, plo_ref, phi_ref):
        st = st_ref[...]
        mu = st[0:1] * (1.0 / N3)
        var = st[1:2] * (1.0 / N3) - mu * mu
        a = _elu((z_ref[...] - mu) * lax.rsqrt(var + EPS) * g_ref[...]
                 + bt_ref[...])
        a_ref[...] = a
        plo_ref[...] = jnp.dot(a, wlo_ref[...],
                               preferred_element_type=jnp.float32)
        phi_ref[...] = jnp.dot(a, whi_ref[...],
                               preferred_element_type=jnp.float32)

    return pl.pallas_call(
        body,
        grid=(G3,),
        in_specs=[
            pl.BlockSpec((R3, 64), lambda i: (i, 0)),
            pl.BlockSpec((2, 64), lambda i: (0, 0)),
            pl.BlockSpec((1, 64), lambda i: (0, 0)),
            pl.BlockSpec((1, 64), lambda i: (0, 0)),
            pl.BlockSpec((64, 32), lambda i: (0, 0)),
            pl.BlockSpec((64, 32), lambda i: (0, 0)),
        ],
        out_specs=[
            pl.BlockSpec((R3, 64), lambda i: (i, 0)),
            pl.BlockSpec((R3, 32), lambda i: (i, 0)),
            pl.BlockSpec((R3, 32), lambda i: (i, 0)),
        ],
        out_shape=[
            jax.ShapeDtypeStruct((N3, 64), jnp.float32),
            jax.ShapeDtypeStruct((N3, 32), jnp.float32),
            jax.ShapeDtypeStruct((N3, 32), jnp.float32),
        ],
    )(z, stats, gamma, beta, wlo, whi)


def _g3_proj(xin, wlo, whi):
    def body(x_ref, wlo_ref, whi_ref, plo_ref, phi_ref):
        xv = x_ref[...]
        plo_ref[...] = jnp.dot(xv, wlo_ref[...],
                               preferred_element_type=jnp.float32)
        phi_ref[...] = jnp.dot(xv, whi_ref[...],
                               preferred_element_type=jnp.float32)

    return pl.pallas_call(
        body,
        grid=(G3,),
        in_specs=[
            pl.BlockSpec((R3, 64), lambda i: (i, 0)),
            pl.BlockSpec((64, 32), lambda i: (0, 0)),
            pl.BlockSpec((64, 32), lambda i: (0, 0)),
        ],
        out_specs=[
            pl.BlockSpec((R3, 32), lambda i: (i, 0)),
            pl.BlockSpec((R3, 32), lambda i: (i, 0)),
        ],
        out_shape=[
            jax.ShapeDtypeStruct((N3, 32), jnp.float32),
            jax.ShapeDtypeStruct((N3, 32), jnp.float32),
        ],
    )(xin, wlo, whi)


def _g3_final(slo, shi, xin, wrel, wroot, b):
    def body(slo_ref, shi_ref, xin_ref, wrel_ref, wroot_ref, b_ref, o_ref):
        s = jnp.concatenate([slo_ref[0] + slo_ref[1],
                             shi_ref[0] + shi_ref[1]], axis=1)
        o_ref[...] = (jnp.dot(s, wrel_ref[...],
                              preferred_element_type=jnp.float32)
                      + jnp.dot(xin_ref[...], wroot_ref[...],
                                preferred_element_type=jnp.float32)
                      + b_ref[...])

    return pl.pallas_call(
        body,
        grid=(G3,),
        in_specs=[
            pl.BlockSpec((2, R3, 32), lambda i: (0, i, 0)),
            pl.BlockSpec((2, R3, 32), lambda i: (0, i, 0)),
            pl.BlockSpec((R3, 64), lambda i: (i, 0)),
            pl.BlockSpec((64, 128), lambda i: (0, 0)),
            pl.BlockSpec((64, 128), lambda i: (0, 0)),
            pl.BlockSpec((1, 128), lambda i: (0, 0)),
        ],
        out_specs=pl.BlockSpec((R3, 128), lambda i: (i, 0)),
        out_shape=jax.ShapeDtypeStruct((N3, 128), jnp.float32),
    )(slo, shi, xin, wrel, wroot, b)


QB = 512


def _knn(px_t, py):
    def body(px_ref, py_ref, o_ref):
        pyv = py_ref[...]
        d2 = jnp.zeros((QB, N2), jnp.float32)
        for k in range(3):
            diff = pyv[:, k:k + 1] - px_ref[k:k + 1, :]
            d2 = d2 + diff * diff
        m = jnp.min(d2, axis=1, keepdims=True)
        io = lax.broadcasted_iota(jnp.int32, (QB, N2), 1)
        sel = jnp.where(d2 == m, io, jnp.int32(N2))
        o_ref[...] = jnp.min(sel, axis=1, keepdims=True)

    return pl.pallas_call(
        body,
        grid=(pl.cdiv(N3, QB),),
        in_specs=[
            pl.BlockSpec((3, N2), lambda i: (0, 0)),
            pl.BlockSpec((QB, 3), lambda i: (i, 0)),
        ],
        out_specs=pl.BlockSpec((QB, 1), lambda i: (i, 0)),
        out_shape=jax.ShapeDtypeStruct((N3, 1), jnp.int32),
        compiler_params=pltpu.CompilerParams(vmem_limit_bytes=100 * 2**20),
    )(px_t, py)


def _onera(pos):
    p0 = pos[:, 0] - math.tan(math.pi / 6) * pos[:, 1]
    pos = jnp.concatenate([p0[:, None], pos[:, 1:]], axis=1)
    return pos * (1 + (1 / 0.56 - 1) * (pos[:, 1:2] / 1.1963))


def _edge_prep(ei, e, ep, npad, nr, n):
    src = jnp.pad(ei[0].astype(jnp.int32), (0, ep - e)).reshape(-1, 128)
    pad_dst = n + jnp.arange(ep - e, dtype=jnp.int32) % (npad - n)
    dst1 = jnp.concatenate([ei[1].astype(jnp.int32), pad_dst])
    dst = jnp.stack([(dst1 + r * npad).reshape(-1, 128) for r in range(nr)])
    return src, dst


def kernel(x, edge_index_2, edge_index_3, pos_2, pos_3, y, params):
    src2, dst2 = _edge_prep(edge_index_2, E2, E2P, N2P, 1, N2)
    src3, dst3 = _edge_prep(edge_index_3, E3, E3P, N3P, 1, N3)
    agg2_32 = _make_sc_agg(CH2, 8, N2P, 32, 4, 1)
    agg2_64 = _make_sc_agg(CH2, 8, N2P, 64, 4, 1)
    agg3_32 = _make_sc_agg(CH3, 7, N3P, 32, 4, 1)

    p = params
    row = lambda a: a.reshape(1, -1)

    wr, wo, b1 = p['conv1']['Wrel'], p['conv1']['Wroot'], p['conv1']['b']
    y2 = y.reshape(1, 1)

    def k0(xv, p2v, yv, wra, wrp, wry, woa, wop, woy):
        def mix(wa, wp, wy):
            t = jnp.dot(xv, wa, preferred_element_type=jnp.float32)
            for k in range(3):
                t = t + p2v[:, k:k + 1] * wp[k:k + 1, :]
            return t + yv * wy
        return mix(wra, wrp, wry), mix(woa, wop, woy)

    proj1, root1 = _tc(
        k0, [(N2, 64), (N2, 64)], x, pos_2, y2,
        wr[:128], wr[128:131], row(wr[131:].sum(0)),
        wo[:128], wo[128:131], row(wo[131:].sum(0)))

    sp = agg2_64(proj1, src2, dst2)

    wn_first = p['erdb1']['rdb1']['conv1']['Wrel']

    def k1(spv, rootv, bv, gv, btv, wnv):
        x1 = _bn_full(_sum2(spv) + bv + rootv, gv, btv)
        return x1, jnp.dot(x1, wnv, preferred_element_type=jnp.float32)

    x1, pj = _tc(k1, [(N2, 64), (N2, 32)], sp, root1, row(b1),
                 row(p['bn1']['gamma']), row(p['bn1']['beta']), wn_first)

    def rdb_run(pr, r0, pj_in, tail, wnext):
        sp1 = agg2_32(pj_in, src2, dst2)
        c1, c2, c3 = pr['conv1'], pr['conv2'], pr['conv3']

        def ka(spv, r0v, bv, wov, gv, btv, w2a, w2b):
            t1 = _elu(_bn_full(
                _sum2(spv) + bv
                + jnp.dot(r0v, wov, preferred_element_type=jnp.float32),
                gv, btv))
            pn = (jnp.dot(r0v, w2a, preferred_element_type=jnp.float32)
                  + jnp.dot(t1, w2b, preferred_element_type=jnp.float32))
            return t1, pn

        t1, pj2 = _tc(ka, [(N2, 32), (N2, 32)], sp1, r0, row(c1['b']),
                      c1['Wroot'], row(pr['bn1']['gamma']),
                      row(pr['bn1']['beta']),
                      c2['Wrel'][:64], c2['Wrel'][64:])

        sp2 = agg2_32(pj2, src2, dst2)

        def kb(spv, r0v, t1v, bv, woa, wob, gv, btv, w3a, w3b, w3c):
            t2 = _elu(_bn_full(
                _sum2(spv) + bv
                + jnp.dot(r0v, woa, preferred_element_type=jnp.float32)
                + jnp.dot(t1v, wob, preferred_element_type=jnp.float32),
                gv, btv))
            pn = (jnp.dot(r0v, w3a, preferred_element_type=jnp.float32)
                  + jnp.dot(t1v, w3b, preferred_element_type=jnp.float32)
                  + jnp.dot(t2, w3c, preferred_element_type=jnp.float32))
            return t2, pn

        w3 = c3['Wrel']
        t2, pj3 = _tc(
            kb, [(N2, 32), (N2, 64)], sp2, r0, t1, row(c2['b']),
            c2['Wroot'][:64], c2['Wroot'][64:],
            row(pr['bn2']['gamma']), row(pr['bn2']['beta']),
            w3[:64], w3[64:96], w3[96:])

        sp3 = agg2_64(pj3, src2, dst2)
        kind, extras = tail

        def kc(spv, r0v, t1v, t2v, bv, woa, wob, woc, gv, btv, *rest):
            t3 = _elu(_bn_full(
                _sum2(spv) + bv
                + jnp.dot(r0v, woa, preferred_element_type=jnp.float32)
                + jnp.dot(t1v, wob, preferred_element_type=jnp.float32)
                + jnp.dot(t2v, woc, preferred_element_type=jnp.float32),
                gv, btv))
            rdbout = r0v + 0.2 * t3
            if kind == 'rdb1':
                (wnv,) = rest
                carry = r0v + 0.2 * rdbout
            elif kind == 'rdb2':
                a0v, wnv = rest
                carry = a0v + 0.2 * rdbout
            else:
                a0v, x1v = rest
                carry = x1v + 0.2 * (a0v + 0.2 * rdbout)
            if kind == 'final':
                return (carry,)
            return carry, jnp.dot(carry, wnv,
                                  preferred_element_type=jnp.float32)

        common = [sp3, r0, t1, t2, row(c3['b']),
                  c3['Wroot'][:64], c3['Wroot'][64:96], c3['Wroot'][96:],
                  row(pr['bn3']['gamma']), row(pr['bn3']['beta'])]
        if kind == 'final':
            (out,) = _tc(kc, [(N2, 64)], *common, *extras),
            return out, None
        res = _tc(kc, [(N2, 64), (N2, 32)], *common, *extras, wnext)
        return res[0], res[1]

    e1, e2 = p['erdb1'], p['erdb2']
    r0b, pj = rdb_run(e1['rdb1'], x1, pj, ('rdb1', []),
                      e1['rdb2']['conv1']['Wrel'])
    x2, pj = rdb_run(e1['rdb2'], r0b, pj, ('rdb2', [x1]),
                     e2['rdb1']['conv1']['Wrel'])
    r0b, pj = rdb_run(e2['rdb1'], x2, pj, ('rdb1', []),
                      e2['rdb2']['conv1']['Wrel'])
    x3, _ = rdb_run(e2['rdb2'], r0b, pj, ('final', [x2, x1]), None)

    px = _onera(pos_2)
    py = _onera(pos_3)
    idx = _knn(px.T, py)[:, 0]
    idxm = jnp.pad(idx, (0, BQ - N3)).reshape(-1, 128)
    xi = _make_sc_gather(CHG, 64, 2)(x3, idxm)[:N3]

    c3p, c4p, c5p = p['conv3'], p['conv4'], p['conv5']
    plo, phi = _g3_proj(xi, c3p['Wrel'][:, :32], c3p['Wrel'][:, 32:])
    slo = agg3_32(plo, src3, dst3)
    shi = agg3_32(phi, src3, dst3)
    z, st = _g3_z_stats(slo, shi, xi, c3p['Wroot'], row(c3p['b']))
    a3, plo, phi = _g3_bn_act(z, st, row(p['bn3']['gamma']),
                              row(p['bn3']['beta']),
                              c4p['Wrel'][:, :32], c4p['Wrel'][:, 32:])
    slo = agg3_32(plo, src3, dst3)
    shi = agg3_32(phi, src3, dst3)
    z, st = _g3_z_stats(slo, shi, a3, c4p['Wroot'], row(c4p['b']))
    eye = jnp.eye(64, dtype=jnp.float32)
    a4, alo, ahi = _g3_bn_act(z, st, row(p['bn4']['gamma']),
                              row(p['bn4']['beta']),
                              eye[:, :32], eye[:, 32:])
    slo = agg3_32(alo, src3, dst3)
    shi = agg3_32(ahi, src3, dst3)
    return _g3_final(slo, shi, a4, c5p['Wrel'], c5p['Wroot'], row(c5p['b']))
